# baseline (speedup 1.0000x reference)
import functools

import jax
import jax.numpy as jnp
from jax.experimental import pallas as pl
from jax.experimental.pallas import tpu as pltpu


def _pass_a(h_ref, emb_ref, pw_ref, pb_ref, dv_ref, de_ref, w1_ref,
            x0_ref, h8_ref, m1_ref, u0t_ref, *, last):
    i = pl.program_id(0)
    x0 = jnp.dot(emb_ref[...].astype(jnp.bfloat16),
                 pw_ref[0].astype(jnp.bfloat16),
                 preferred_element_type=jnp.float32) + pb_ref[0]
    x0_ref[...] = x0.astype(jnp.bfloat16)
    h = h_ref[...]
    h8_ref[...] = h.astype(jnp.float8_e4m3fn)
    xs = (x0 * dv_ref[...]).astype(jnp.bfloat16)
    contrib = jax.lax.dot_general(
        xs, h.astype(jnp.bfloat16), (((0,), (0,)), ((), ())),
        preferred_element_type=jnp.float32)

    @pl.when(i == 0)
    def _():
        u0t_ref[...] = contrib

    @pl.when(i > 0)
    def _():
        u0t_ref[...] += contrib

    @pl.when(i == last)
    def _():
        m0t = (u0t_ref[...] * de_ref[...]).astype(jnp.bfloat16)
        m1_ref[...] = jax.lax.dot_general(
            w1_ref[...].astype(jnp.bfloat16), m0t, (((0,), (0,)), ((), ())),
            preferred_element_type=jnp.float32).astype(jnp.bfloat16)


def _pass_b(h8_ref, m1_ref, dv_ref, b1_ref, x1_ref):
    hb = h8_ref[...].astype(jnp.bfloat16)
    hm = jax.lax.dot_general(
        hb, m1_ref[...], (((1,), (1,)), ((), ())),
        preferred_element_type=jnp.float32)
    x1_ref[...] = (hm * dv_ref[...] + b1_ref[...]).astype(jnp.bfloat16)


def _pass_d(h8_ref, x1_ref, dv_ref, de_ref, w2_ref, m2_ref, u1t_ref,
            *, last):
    i = pl.program_id(0)
    hb = h8_ref[...].astype(jnp.bfloat16)
    xs1 = (x1_ref[...].astype(jnp.float32) * dv_ref[...]).astype(jnp.bfloat16)
    contrib = jax.lax.dot_general(
        xs1, hb, (((0,), (0,)), ((), ())),
        preferred_element_type=jnp.float32)

    @pl.when(i == 0)
    def _():
        u1t_ref[...] = contrib

    @pl.when(i > 0)
    def _():
        u1t_ref[...] += contrib

    @pl.when(i == last)
    def _():
        m1t = (u1t_ref[...] * de_ref[...]).astype(jnp.bfloat16)
        m2_ref[...] = jax.lax.dot_general(
            w2_ref[...].astype(jnp.bfloat16), m1t, (((0,), (0,)), ((), ())),
            preferred_element_type=jnp.float32).astype(jnp.bfloat16)


def _pass_c(fus_ref, h8_ref, m2_ref, dv_ref, b2_ref,
            x0_ref, x1_ref, out_ref):
    hb = h8_ref[...].astype(jnp.bfloat16)
    hm = jax.lax.dot_general(
        hb, m2_ref[...], (((1,), (1,)), ((), ())),
        preferred_element_type=jnp.float32)
    x2 = hm * dv_ref[...] + b2_ref[...]
    f0, f1, f2 = fus_ref[0], fus_ref[1], fus_ref[2]
    mx = jnp.maximum(jnp.maximum(f0, f1), f2)
    e0, e1, e2 = jnp.exp(f0 - mx), jnp.exp(f1 - mx), jnp.exp(f2 - mx)
    s = e0 + e1 + e2
    out_ref[...] = ((e0 / s) * x0_ref[...].astype(jnp.float32)
                    + (e1 / s) * x1_ref[...].astype(jnp.float32) + (e2 / s) * x2)


@functools.partial(jax.jit, static_argnames=())
def kernel(H, Dv_inv_sqrt, De_inv, emb, projW, projB, W1, b1, W2, b2, fusion):
    N, E = H.shape
    D = emb.shape[1]
    T = projW.shape[0]
    per_t = N // T
    NB = 400 if per_t % 400 == 0 else per_t
    nblk = N // NB
    blk_per_t = per_t // NB
    NB2 = 1000 if N % 2000 == 0 else NB
    nblk2 = N // NB2

    dv = Dv_inv_sqrt.reshape(N, 1)
    de_row = De_inv.reshape(1, E)
    b1r = b1.reshape(1, D)
    b2r = b2.reshape(1, D)

    x0, h8, m1 = pl.pallas_call(
        functools.partial(_pass_a, last=nblk - 1),
        grid=(nblk,),
        in_specs=[
            pl.BlockSpec((NB, E), lambda i: (i, 0)),
            pl.BlockSpec((NB, D), lambda i: (i, 0)),
            pl.BlockSpec((1, D, D), lambda i, _b=blk_per_t: (i // _b, 0, 0)),
            pl.BlockSpec((1, 1, D), lambda i, _b=blk_per_t: (i // _b, 0, 0)),
            pl.BlockSpec((NB, 1), lambda i: (i, 0)),
            pl.BlockSpec((1, E), lambda i: (0, 0)),
            pl.BlockSpec((D, D), lambda i: (0, 0)),
        ],
        out_specs=[
            pl.BlockSpec((NB, D), lambda i: (i, 0)),
            pl.BlockSpec((NB, E), lambda i: (i, 0)),
            pl.BlockSpec((D, E), lambda i: (0, 0)),
        ],
        out_shape=[
            jax.ShapeDtypeStruct((N, D), jnp.bfloat16),
            jax.ShapeDtypeStruct((N, E), jnp.float8_e4m3fn),
            jax.ShapeDtypeStruct((D, E), jnp.bfloat16),
        ],
        scratch_shapes=[pltpu.VMEM((D, E), jnp.float32)],
    )(H, emb, projW, projB.reshape(T, 1, D), dv, de_row, W1)

    x1 = pl.pallas_call(
        _pass_b,
        grid=(nblk2,),
        in_specs=[
            pl.BlockSpec((NB2, E), lambda i: (i, 0)),
            pl.BlockSpec((D, E), lambda i: (0, 0)),
            pl.BlockSpec((NB2, 1), lambda i: (i, 0)),
            pl.BlockSpec((1, D), lambda i: (0, 0)),
        ],
        out_specs=pl.BlockSpec((NB2, D), lambda i: (i, 0)),
        out_shape=jax.ShapeDtypeStruct((N, D), jnp.bfloat16),
    )(h8, m1, dv, b1r)

    m2 = pl.pallas_call(
        functools.partial(_pass_d, last=nblk2 - 1),
        grid=(nblk2,),
        in_specs=[
            pl.BlockSpec((NB2, E), lambda i: (i, 0)),
            pl.BlockSpec((NB2, D), lambda i: (i, 0)),
            pl.BlockSpec((NB2, 1), lambda i: (i, 0)),
            pl.BlockSpec((1, E), lambda i: (0, 0)),
            pl.BlockSpec((D, D), lambda i: (0, 0)),
        ],
        out_specs=pl.BlockSpec((D, E), lambda i: (0, 0)),
        out_shape=jax.ShapeDtypeStruct((D, E), jnp.bfloat16),
        scratch_shapes=[pltpu.VMEM((D, E), jnp.float32)],
    )(h8, x1, dv, de_row, W2)

    out = pl.pallas_call(
        _pass_c,
        grid=(nblk2,),
        in_specs=[
            pl.BlockSpec(memory_space=pltpu.SMEM),
            pl.BlockSpec((NB2, E), lambda i: (i, 0)),
            pl.BlockSpec((D, E), lambda i: (0, 0)),
            pl.BlockSpec((NB2, 1), lambda i: (i, 0)),
            pl.BlockSpec((1, D), lambda i: (0, 0)),
            pl.BlockSpec((NB2, D), lambda i: (i, 0)),
            pl.BlockSpec((NB2, D), lambda i: (i, 0)),
        ],
        out_specs=pl.BlockSpec((NB2, D), lambda i: (i, 0)),
        out_shape=jax.ShapeDtypeStruct((N, D), jnp.float32),
    )(fusion, h8, m2, dv, b2r, x0, x1)

    return out


# speedup vs baseline: 1.0172x; 1.0172x over previous
import functools

import jax
import jax.numpy as jnp
from jax.experimental import pallas as pl
from jax.experimental.pallas import tpu as pltpu


def _pow2_scale(mx):
    mx = jnp.maximum(mx, 1e-30)
    e = ((jax.lax.bitcast_convert_type(mx, jnp.int32) >> 23) & 0xFF) - 127
    s = jax.lax.bitcast_convert_type(((134 - e) << 23).astype(jnp.int32),
                                     jnp.float32)
    inv_s = jax.lax.bitcast_convert_type(((120 + e) << 23).astype(jnp.int32),
                                         jnp.float32)
    return s, inv_s


def _hi_lo_f8(m, s):
    ms = m * s
    hi = ms.astype(jnp.float8_e4m3fn)
    lo = (ms - hi.astype(jnp.float32)).astype(jnp.float8_e4m3fn)
    return hi, lo

def _pass_a(h_ref, emb_ref, pw_ref, pb_ref, dv_ref, de_ref, w1_ref,
            x0_ref, h8_ref, m1c_ref, s1_ref, u0t_ref, *, last):
    i = pl.program_id(0)
    x0 = jnp.dot(emb_ref[...].astype(jnp.bfloat16),
                 pw_ref[0].astype(jnp.bfloat16),
                 preferred_element_type=jnp.float32) + pb_ref[0]
    x0_ref[...] = x0.astype(jnp.bfloat16)
    h = h_ref[...]
    h8_ref[...] = h.astype(jnp.float8_e4m3fn)
    xs = (x0 * dv_ref[...]).astype(jnp.bfloat16)
    contrib = jax.lax.dot_general(
        xs, h.astype(jnp.bfloat16), (((0,), (0,)), ((), ())),
        preferred_element_type=jnp.float32)

    @pl.when(i == 0)
    def _():
        u0t_ref[...] = contrib

    @pl.when(i > 0)
    def _():
        u0t_ref[...] += contrib

    @pl.when(i == last)
    def _():
        m0t = (u0t_ref[...] * de_ref[...]).astype(jnp.bfloat16)
        m1f = jax.lax.dot_general(
            w1_ref[...].astype(jnp.bfloat16), m0t, (((0,), (0,)), ((), ())),
            preferred_element_type=jnp.float32)
        s, inv_s = _pow2_scale(jnp.max(jnp.abs(m1f)))
        hi, lo = _hi_lo_f8(m1f, s)
        m1c_ref[...] = jnp.concatenate([hi, lo], axis=0)
        s1_ref[...] = jnp.full((1, 128), inv_s, jnp.float32)


def _pass_b(h8_ref, m1c_ref, s1_ref, dv_ref, b1_ref, x1_ref):
    d = m1c_ref.shape[0] // 2
    hm2 = jax.lax.dot_general(
        h8_ref[...], m1c_ref[...], (((1,), (1,)), ((), ())),
        preferred_element_type=jnp.float32)
    hm = hm2[:, :d] + hm2[:, d:]
    x1_ref[...] = (hm * (dv_ref[...] * s1_ref[0, 0]) + b1_ref[...]).astype(jnp.bfloat16)


def _pass_d(h8_ref, x1_ref, dv_ref, de_ref, w2_ref, m2c_ref, s2_ref,
            u1t_ref, *, last):
    i = pl.program_id(0)
    hb = h8_ref[...].astype(jnp.bfloat16)
    xs1 = (x1_ref[...].astype(jnp.float32) * dv_ref[...]).astype(jnp.bfloat16)
    contrib = jax.lax.dot_general(
        xs1, hb, (((0,), (0,)), ((), ())),
        preferred_element_type=jnp.float32)

    @pl.when(i == 0)
    def _():
        u1t_ref[...] = contrib

    @pl.when(i > 0)
    def _():
        u1t_ref[...] += contrib

    @pl.when(i == last)
    def _():
        m1t = (u1t_ref[...] * de_ref[...]).astype(jnp.bfloat16)
        m2f = jax.lax.dot_general(
            w2_ref[...].astype(jnp.bfloat16), m1t, (((0,), (0,)), ((), ())),
            preferred_element_type=jnp.float32)
        s, inv_s = _pow2_scale(jnp.max(jnp.abs(m2f)))
        hi, lo = _hi_lo_f8(m2f, s)
        m2c_ref[...] = jnp.concatenate([hi, lo], axis=0)
        s2_ref[...] = jnp.full((1, 128), inv_s, jnp.float32)


def _pass_c(fus_ref, h8_ref, m2c_ref, s2_ref, dv_ref, b2_ref,
            x0_ref, x1_ref, out_ref):
    d = x0_ref.shape[1]
    hm2 = jax.lax.dot_general(
        h8_ref[...], m2c_ref[...], (((1,), (1,)), ((), ())),
        preferred_element_type=jnp.float32)
    hm = hm2[:, :d] + hm2[:, d:]
    x2 = hm * (dv_ref[...] * s2_ref[0, 0]) + b2_ref[...]
    f0, f1, f2 = fus_ref[0], fus_ref[1], fus_ref[2]
    mx = jnp.maximum(jnp.maximum(f0, f1), f2)
    e0, e1, e2 = jnp.exp(f0 - mx), jnp.exp(f1 - mx), jnp.exp(f2 - mx)
    s = e0 + e1 + e2
    out_ref[...] = ((e0 / s) * x0_ref[...].astype(jnp.float32)
                    + (e1 / s) * x1_ref[...].astype(jnp.float32) + (e2 / s) * x2)


@functools.partial(jax.jit, static_argnames=())
def kernel(H, Dv_inv_sqrt, De_inv, emb, projW, projB, W1, b1, W2, b2, fusion):
    N, E = H.shape
    D = emb.shape[1]
    T = projW.shape[0]
    per_t = N // T
    NB = 400 if per_t % 400 == 0 else per_t
    nblk = N // NB
    blk_per_t = per_t // NB
    NB2 = 2000 if N % 2000 == 0 else NB
    nblk2 = N // NB2

    dv = Dv_inv_sqrt.reshape(N, 1)
    de_row = De_inv.reshape(1, E)
    b1r = b1.reshape(1, D)
    b2r = b2.reshape(1, D)

    x0, h8, m1c, s1 = pl.pallas_call(
        functools.partial(_pass_a, last=nblk - 1),
        grid=(nblk,),
        in_specs=[
            pl.BlockSpec((NB, E), lambda i: (i, 0)),
            pl.BlockSpec((NB, D), lambda i: (i, 0)),
            pl.BlockSpec((1, D, D), lambda i, _b=blk_per_t: (i // _b, 0, 0)),
            pl.BlockSpec((1, 1, D), lambda i, _b=blk_per_t: (i // _b, 0, 0)),
            pl.BlockSpec((NB, 1), lambda i: (i, 0)),
            pl.BlockSpec((1, E), lambda i: (0, 0)),
            pl.BlockSpec((D, D), lambda i: (0, 0)),
        ],
        out_specs=[
            pl.BlockSpec((NB, D), lambda i: (i, 0)),
            pl.BlockSpec((NB, E), lambda i: (i, 0)),
            pl.BlockSpec((2 * D, E), lambda i: (0, 0)),
            pl.BlockSpec((1, D), lambda i: (0, 0)),
        ],
        out_shape=[
            jax.ShapeDtypeStruct((N, D), jnp.bfloat16),
            jax.ShapeDtypeStruct((N, E), jnp.float8_e4m3fn),
            jax.ShapeDtypeStruct((2 * D, E), jnp.float8_e4m3fn),
            jax.ShapeDtypeStruct((1, D), jnp.float32),
        ],
        scratch_shapes=[pltpu.VMEM((D, E), jnp.float32)],
    )(H, emb, projW, projB.reshape(T, 1, D), dv, de_row, W1)

    x1 = pl.pallas_call(
        _pass_b,
        grid=(nblk2,),
        in_specs=[
            pl.BlockSpec((NB2, E), lambda i: (i, 0)),
            pl.BlockSpec((2 * D, E), lambda i: (0, 0)),
            pl.BlockSpec((1, D), lambda i: (0, 0)),
            pl.BlockSpec((NB2, 1), lambda i: (i, 0)),
            pl.BlockSpec((1, D), lambda i: (0, 0)),
        ],
        out_specs=pl.BlockSpec((NB2, D), lambda i: (i, 0)),
        out_shape=jax.ShapeDtypeStruct((N, D), jnp.bfloat16),
    )(h8, m1c, s1, dv, b1r)

    m2c, s2 = pl.pallas_call(
        functools.partial(_pass_d, last=nblk2 - 1),
        grid=(nblk2,),
        in_specs=[
            pl.BlockSpec((NB2, E), lambda i: (i, 0)),
            pl.BlockSpec((NB2, D), lambda i: (i, 0)),
            pl.BlockSpec((NB2, 1), lambda i: (i, 0)),
            pl.BlockSpec((1, E), lambda i: (0, 0)),
            pl.BlockSpec((D, D), lambda i: (0, 0)),
        ],
        out_specs=[
            pl.BlockSpec((2 * D, E), lambda i: (0, 0)),
            pl.BlockSpec((1, D), lambda i: (0, 0)),
        ],
        out_shape=[
            jax.ShapeDtypeStruct((2 * D, E), jnp.float8_e4m3fn),
            jax.ShapeDtypeStruct((1, D), jnp.float32),
        ],
        scratch_shapes=[pltpu.VMEM((D, E), jnp.float32)],
    )(h8, x1, dv, de_row, W2)

    out = pl.pallas_call(
        _pass_c,
        grid=(nblk2,),
        in_specs=[
            pl.BlockSpec(memory_space=pltpu.SMEM),
            pl.BlockSpec((NB2, E), lambda i: (i, 0)),
            pl.BlockSpec((2 * D, E), lambda i: (0, 0)),
            pl.BlockSpec((1, D), lambda i: (0, 0)),
            pl.BlockSpec((NB2, 1), lambda i: (i, 0)),
            pl.BlockSpec((1, D), lambda i: (0, 0)),
            pl.BlockSpec((NB2, D), lambda i: (i, 0)),
            pl.BlockSpec((NB2, D), lambda i: (i, 0)),
        ],
        out_specs=pl.BlockSpec((NB2, D), lambda i: (i, 0)),
        out_shape=jax.ShapeDtypeStruct((N, D), jnp.float32),
    )(fusion, h8, m2c, s2, dv, b2r, x0, x1)

    return out
